# baseline (speedup 1.0000x reference)
import functools

import jax
import jax.numpy as jnp
from jax import lax
from jax.experimental import pallas as pl
from jax.experimental.pallas import tpu as pltpu
from jax.experimental.pallas import tpu_sc as plsc

SEQ = 200
BATCH = 1024
DIM = 64
N = SEQ * BATCH
NC = 2
NS = 16
NW = NC * NS
ROWS_W = N // NW
G = 128
NG = ROWS_W // G
NBUF = 5
NITER = NG // NBUF
LANES = 16


def _emb_body(table_hbm, idx_hbm, out_hbm, idx_v, rows_v, gsem, ssem):
    wid = lax.axis_index("s") * NC + lax.axis_index("c")
    base = wid * ROWS_W

    pltpu.sync_copy(idx_hbm.at[wid], idx_v)

    def gather_desc(g, slot):
        return pltpu.make_async_copy(
            table_hbm.at[idx_v.at[g]], rows_v.at[slot], gsem.at[slot])

    def store_desc(g, slot):
        return pltpu.make_async_copy(
            rows_v.at[slot], out_hbm.at[pl.ds(base + g * G, G)], ssem.at[slot])

    for b in range(NBUF - 1):
        gather_desc(b, b).start()

    @pl.loop(0, NITER)
    def _outer(i):
        for b in range(NBUF):
            h = i * NBUF + b
            slot = b
            gather_desc(h, slot).wait()

            cnt_vec = jnp.zeros((LANES,), jnp.int32)
            for k in range(G // LANES):
                v16 = idx_v[h, pl.ds(k * LANES, LANES)]
                cnt_vec = cnt_vec + jnp.where(v16 == 0, 1, 0).astype(jnp.int32)
            cnt = jnp.sum(cnt_vec)

            @pl.when(cnt > 0)
            def _fix():
                @pl.loop(0, G // LANES)
                def _per16(k):
                    v16 = idx_v[h, pl.ds(k * LANES, LANES)]
                    scale = jnp.where(v16 == 0, jnp.float32(0), jnp.float32(1))

                    @pl.loop(0, LANES)
                    def _per_row(r):
                        lane = jnp.broadcast_to(r, (LANES,)).astype(jnp.int32)
                        srow = lax.gather(
                            scale, lane[:, None],
                            lax.GatherDimensionNumbers(
                                offset_dims=(), collapsed_slice_dims=(0,),
                                start_index_map=(0,)),
                            slice_sizes=(1,),
                            mode=lax.GatherScatterMode.PROMISE_IN_BOUNDS)
                        row = k * LANES + r
                        for c in range(DIM // LANES):
                            sl = pl.ds(c * LANES, LANES)
                            rows_v[slot, row, sl] = rows_v[slot, row, sl] * srow

            store_desc(h, slot).start()

            p = h + NBUF - 1
            slot_p = (b - 1) % NBUF
            if b == 0:
                @pl.when(p < NG)
                def _refill0():
                    @pl.when(i >= 1)
                    def _drain_prev():
                        store_desc(h - 1, slot_p).wait()
                    gather_desc(p, slot_p).start()
            else:
                @pl.when(p < NG)
                def _refill():
                    store_desc(h - 1, slot_p).wait()
                    gather_desc(p, slot_p).start()

    for b in range(NBUF):
        store_desc(NG - NBUF + b, b).wait()


_emb_lookup = functools.partial(
    pl.kernel,
    out_type=jax.ShapeDtypeStruct((N, DIM), jnp.float32),
    mesh=plsc.VectorSubcoreMesh(
        core_axis_name="c", subcore_axis_name="s",
        num_cores=NC, num_subcores=NS),
    scratch_types=[
        pltpu.VMEM((NG, G), jnp.int32),
        pltpu.VMEM((NBUF, G, DIM), jnp.float32),
        pltpu.SemaphoreType.DMA((NBUF,)),
        pltpu.SemaphoreType.DMA((NBUF,)),
    ],
    compiler_params=pltpu.CompilerParams(
        needs_layout_passes=False, use_tc_tiling_on_sc=False),
)(_emb_body)


@jax.jit
def kernel(src_input, table):
    idx = src_input.reshape(N).astype(jnp.int32).reshape(NW, NG, G)
    out = _emb_lookup(table, idx)
    return out.reshape(SEQ, BATCH, DIM)
